# baseline (speedup 1.0000x reference)
import functools

import jax
import jax.numpy as jnp
from jax import lax
from jax.experimental import pallas as pl
from jax.experimental.pallas import tpu as pltpu
from jax.experimental.pallas import tpu_sc as plsc

N = 10000
E = 320000
D = 128
H = 64
W = 80
NC = 2
NS = 16
NW = NC * NS
CH = 128
NCHUNK = E // CH
NFULL = NCHUNK // NW
NEXTRA = NCHUNK - NFULL * NW
NPAD = 10240
ROWS_PER_TILE = NPAD // NS

assert NFULL % 2 == 0 and ROWS_PER_TILE % CH == 0


def _node_proj_body(nf_ref, wsT_ref, wrT_ref, b_ref, ps_ref, pr_ref):
    nf = nf_ref[...]
    ps_ref[...] = jnp.dot(nf, wsT_ref[...], preferred_element_type=jnp.float32,
                          precision=jax.lax.Precision.HIGHEST) + b_ref[...]
    pr_ref[...] = jnp.dot(nf, wrT_ref[...], preferred_element_type=jnp.float32,
                          precision=jax.lax.Precision.HIGHEST)


def _node_proj(nf, wsT, wrT, bm1):
    return pl.pallas_call(
        _node_proj_body,
        out_shape=(
            jax.ShapeDtypeStruct((N, H), jnp.float32),
            jax.ShapeDtypeStruct((N, H), jnp.float32),
        ),
    )(nf, wsT, wrT, bm1.reshape(1, H))


def _sc_body(ps_hbm, pr_hbm, ei_hbm, ef_hbm, wp_hbm, out_hbm,
             acc, wv,
             sidx_a, ridx_a, hs_a, hr_a, efv_a, msg_a,
             sidx_b, ridx_b, hs_b, hr_b, efv_b, msg_b,
             sem_sa, sem_ra, sem_ea, sem_sb, sem_rb, sem_eb):
    cid = lax.axis_index("c")
    sid = lax.axis_index("s")
    wid = sid * NC + cid

    def zero_row(i, _):
        for k in range(W // 16):
            msg_a[i, pl.ds(k * 16, 16)] = jnp.zeros((16,), jnp.float32)
            msg_b[i, pl.ds(k * 16, 16)] = jnp.zeros((16,), jnp.float32)
        return 0

    lax.fori_loop(0, CH, zero_row, 0)
    for k in range(ROWS_PER_TILE // CH):
        pltpu.sync_copy(msg_a, acc.at[pl.ds(sid * ROWS_PER_TILE + k * CH, CH),
                                      pl.ds(0, W)])
    plsc.subcore_barrier()

    one_lane = jnp.where(lax.iota(jnp.int32, 16) == 0,
                         jnp.float32(1.0), jnp.float32(0.0))

    def one_row(i, _):
        msg_a[i, pl.ds(H, 16)] = one_lane
        msg_b[i, pl.ds(H, 16)] = one_lane
        return 0

    lax.fori_loop(0, CH, one_row, 0)
    pltpu.sync_copy(wp_hbm, wv)
    wk = [[wv[f, pl.ds(k * 16, 16)] for k in range(H // 16)] for f in range(4)]

    def start(t, sidx, ridx, hs, hr, efv, sem_s, sem_r, sem_e):
        off = (t * NW + wid) * CH
        pltpu.sync_copy(ei_hbm.at[0, pl.ds(off, CH)], sidx)
        pltpu.sync_copy(ei_hbm.at[1, pl.ds(off, CH)], ridx)
        pltpu.async_copy(ps_hbm.at[sidx], hs, sem_s)
        pltpu.async_copy(pr_hbm.at[ridx], hr, sem_r)
        pltpu.async_copy(ef_hbm.at[pl.ds(off * 4, CH * 4)], efv, sem_e)

    def wait(hs, hr, efv, sem_s, sem_r, sem_e):
        pltpu.make_async_copy(ps_hbm.at[sidx_a], hs, sem_s).wait()
        pltpu.make_async_copy(pr_hbm.at[ridx_a], hr, sem_r).wait()
        pltpu.make_async_copy(ef_hbm.at[pl.ds(0, CH * 4)], efv, sem_e).wait()

    def compute(hs, hr, efv, msg):
        def quad(q, _):
            vblk = efv[pl.ds(q * 16, 16)]
            for e4 in range(4):
                i = q * 4 + e4
                eb = [vblk.at[jnp.full((16,), 4 * e4 + f, jnp.int32)]
                      .get(mode="promise_in_bounds") for f in range(4)]
                for k in range(H // 16):
                    sl = pl.ds(k * 16, 16)
                    z = hs[i, sl] + hr[i, sl]
                    for f in range(4):
                        z = z + eb[f] * wk[f][k]
                    msg[i, sl] = jnp.where(z > 0.0, z, jnp.exp(z) - 1.0)
            return 0

        lax.fori_loop(0, CH // 4, quad, 0)

    def scatter(msg, ridx):
        pltpu.sync_copy(msg, acc.at[ridx], add=True)

    start(0, sidx_a, ridx_a, hs_a, hr_a, efv_a, sem_sa, sem_ra, sem_ea)

    def pair(j, _):
        wait(hs_a, hr_a, efv_a, sem_sa, sem_ra, sem_ea)
        start(2 * j + 1, sidx_b, ridx_b, hs_b, hr_b, efv_b,
              sem_sb, sem_rb, sem_eb)
        compute(hs_a, hr_a, efv_a, msg_a)
        scatter(msg_a, ridx_a)

        wait(hs_b, hr_b, efv_b, sem_sb, sem_rb, sem_eb)

        @pl.when(j < NFULL // 2 - 1)
        def _():
            start(2 * j + 2, sidx_a, ridx_a, hs_a, hr_a, efv_a,
                  sem_sa, sem_ra, sem_ea)

        compute(hs_b, hr_b, efv_b, msg_b)
        scatter(msg_b, ridx_b)
        return 0

    lax.fori_loop(0, NFULL // 2, pair, 0)

    @pl.when(wid < NEXTRA)
    def _():
        start(NFULL, sidx_a, ridx_a, hs_a, hr_a, efv_a, sem_sa, sem_ra, sem_ea)
        wait(hs_a, hr_a, efv_a, sem_sa, sem_ra, sem_ea)
        compute(hs_a, hr_a, efv_a, msg_a)
        scatter(msg_a, ridx_a)

    plsc.subcore_barrier()

    pltpu.sync_copy(
        acc.at[pl.ds(sid * ROWS_PER_TILE, ROWS_PER_TILE), :],
        out_hbm.at[cid, pl.ds(sid * ROWS_PER_TILE, ROWS_PER_TILE), :],
    )


_sc_gather_scatter = functools.partial(
    pl.kernel,
    out_type=jax.ShapeDtypeStruct((NC, NPAD, W), jnp.float32),
    mesh=plsc.VectorSubcoreMesh(core_axis_name="c", subcore_axis_name="s",
                                num_cores=NC, num_subcores=NS),
    scratch_types=[
        pltpu.VMEM_SHARED((NPAD, W), jnp.float32),
        pltpu.VMEM((4, H), jnp.float32),
        pltpu.VMEM((CH,), jnp.int32),
        pltpu.VMEM((CH,), jnp.int32),
        pltpu.VMEM((CH, H), jnp.float32),
        pltpu.VMEM((CH, H), jnp.float32),
        pltpu.VMEM((CH * 4,), jnp.float32),
        pltpu.VMEM((CH, W), jnp.float32),
        pltpu.VMEM((CH,), jnp.int32),
        pltpu.VMEM((CH,), jnp.int32),
        pltpu.VMEM((CH, H), jnp.float32),
        pltpu.VMEM((CH, H), jnp.float32),
        pltpu.VMEM((CH * 4,), jnp.float32),
        pltpu.VMEM((CH, W), jnp.float32),
        pltpu.SemaphoreType.DMA,
        pltpu.SemaphoreType.DMA,
        pltpu.SemaphoreType.DMA,
        pltpu.SemaphoreType.DMA,
        pltpu.SemaphoreType.DMA,
        pltpu.SemaphoreType.DMA,
    ],
    compiler_params=pltpu.CompilerParams(use_tc_tiling_on_sc=False),
)(_sc_body)


def _post_body(nf_ref, p0_ref, p1_ref, wm2T_ref, wu1lT_ref, wu1rT_ref,
               bm2_ref, bu1_ref, wu2T_ref, bu2_ref, out_ref):
    p0 = p0_ref[...]
    p1 = p1_ref[...]
    s = p0[:, :H] + p1[:, :H]
    deg = p0[:, H:H + 1] + p1[:, H:H + 1]
    wcT = jnp.dot(wm2T_ref[...], wu1rT_ref[...],
                  preferred_element_type=jnp.float32,
                  precision=jax.lax.Precision.HIGHEST)
    bvec = jnp.dot(bm2_ref[...], wu1rT_ref[...],
                   preferred_element_type=jnp.float32,
                   precision=jax.lax.Precision.HIGHEST)
    u = (jnp.dot(nf_ref[...], wu1lT_ref[...], preferred_element_type=jnp.float32,
                 precision=jax.lax.Precision.HIGHEST)
         + jnp.dot(s, wcT, preferred_element_type=jnp.float32,
                   precision=jax.lax.Precision.HIGHEST)
         + deg * bvec
         + bu1_ref[...])
    h2 = jnp.where(u > 0.0, u, jnp.exp(u) - 1.0)
    out_ref[...] = (jnp.dot(h2, wu2T_ref[...], preferred_element_type=jnp.float32,
                            precision=jax.lax.Precision.HIGHEST)
                    + bu2_ref[...])


def _post(nf, p0, p1, wm2T, wu1lT, wu1rT, bm2, bu1, wu2T, bu2):
    BN = 1000
    grid = N // BN
    wspec = lambda shape: pl.BlockSpec(shape, lambda i: (0, 0))
    return pl.pallas_call(
        _post_body,
        grid=(grid,),
        in_specs=[
            pl.BlockSpec((BN, D), lambda i: (i, 0)),
            pl.BlockSpec((BN, W), lambda i: (i, 0)),
            pl.BlockSpec((BN, W), lambda i: (i, 0)),
            wspec((H, H)),
            wspec((D, H)),
            wspec((H, H)),
            wspec((1, H)),
            wspec((1, H)),
            wspec((H, D)),
            wspec((1, D)),
        ],
        out_specs=pl.BlockSpec((BN, D), lambda i: (i, 0)),
        out_shape=jax.ShapeDtypeStruct((N, D), jnp.float32),
    )(nf, p0, p1, wm2T, wu1lT, wu1rT, bm2.reshape(1, H), bu1.reshape(1, H),
      wu2T, bu2.reshape(1, D))


def kernel(node_features, edge_index, edge_features,
           Wm1, bm1, Wm2, bm2, Wu1, bu1, Wu2, bu2):
    wsT = Wm1[:, :D].T
    wrT = Wm1[:, D:2 * D].T
    weT = Wm1[:, 2 * D:].T

    ps, pr = _node_proj(node_features, wsT, wrT, bm1)
    partials = _sc_gather_scatter(ps, pr, edge_index,
                                  edge_features.reshape(-1), weT)

    return _post(
        node_features,
        partials[0],
        partials[1],
        Wm2.T,
        Wu1[:, :D].T,
        Wu1[:, D:].T,
        bm2,
        bu1,
        Wu2.T,
        bu2,
    )
